# Initial kernel scaffold; baseline (speedup 1.0000x reference)
import functools

import jax
import jax.numpy as jnp
from jax import lax
from jax.experimental import pallas as pl
from jax.experimental.pallas import tpu as pltpu
from jax.experimental.pallas import tpu_sc as plsc

NU = 25000
NI = 25000
NNODES = NU + NI
K = 64
NE = 400000
BATCH = 16384

NC = 2
NS = 16
EPW = NE // NS
CH = 1000
NCH = EPW // CH
HALF = NU
RPW = 1564
SP_ROWS = RPW * NS
TAIL = HALF - (NS - 1) * RPW

BPW = BATCH // (NC * NS)

_MESH = plsc.VectorSubcoreMesh(core_axis_name="c", subcore_axis_name="s")


def _spmv_body(src_hbm, dst_hbm, x_hbm, zero_hbm, out_hbm,
               sidx_v, didx_v, rows_v, s_sh, sem):
    c = lax.axis_index("c")
    t = lax.axis_index("s")

    pltpu.sync_copy(zero_hbm, s_sh.at[pl.ds(t * RPW, RPW)])
    plsc.subcore_barrier()

    ebase = t * EPW

    def chunk(g, carry):
        off = ebase + g * CH
        pltpu.sync_copy(src_hbm.at[c, pl.ds(off, CH)], sidx_v)
        pltpu.sync_copy(dst_hbm.at[c, pl.ds(off, CH)], didx_v)
        pltpu.async_copy(x_hbm.at[sidx_v], rows_v, sem).wait()
        pltpu.sync_copy(rows_v, s_sh.at[didx_v], add=True)
        return carry

    lax.fori_loop(0, NCH, chunk, 0)
    plsc.subcore_barrier()

    rbase = t * RPW

    @pl.when(t < NS - 1)
    def _():
        pltpu.sync_copy(s_sh.at[pl.ds(rbase, RPW)],
                        out_hbm.at[pl.ds(c * HALF + rbase, RPW)])

    @pl.when(t == NS - 1)
    def _():
        pltpu.sync_copy(s_sh.at[pl.ds(rbase, TAIL)],
                        out_hbm.at[pl.ds(c * HALF + rbase, TAIL)])


_spmv = functools.partial(
    pl.kernel,
    out_type=jax.ShapeDtypeStruct((NNODES, K), jnp.float32),
    mesh=_MESH,
    scratch_types=[
        pltpu.VMEM((CH,), jnp.int32),
        pltpu.VMEM((CH,), jnp.int32),
        pltpu.VMEM((CH, K), jnp.float32),
        pltpu.VMEM_SHARED((SP_ROWS, K), jnp.float32),
        pltpu.SemaphoreType.DMA,
    ],
)(_spmv_body)


def _gather_body(t0, t1, t2, t3, uid_hbm, iid_hbm, gu_hbm, gi_hbm,
                 idx_v, rows_v, sem):
    c = lax.axis_index("c")
    t = lax.axis_index("s")
    wid = t * NC + c
    base = wid * BPW

    for ids, out in ((uid_hbm, gu_hbm), (iid_hbm, gi_hbm)):
        pltpu.sync_copy(ids.at[pl.ds(base, BPW)], idx_v)
        for l, tab in enumerate((t0, t1, t2, t3)):
            pltpu.async_copy(tab.at[idx_v], rows_v, sem).wait()
            pltpu.sync_copy(rows_v, out.at[l, pl.ds(base, BPW)])


_gather = functools.partial(
    pl.kernel,
    out_type=(jax.ShapeDtypeStruct((4, BATCH, K), jnp.float32),
              jax.ShapeDtypeStruct((4, BATCH, K), jnp.float32)),
    mesh=_MESH,
    scratch_types=[
        pltpu.VMEM((BPW,), jnp.int32),
        pltpu.VMEM((BPW, K), jnp.float32),
        pltpu.SemaphoreType.DMA,
    ],
)(_gather_body)


BLK = 2000


def _dense_body(x_ref, s_ref, w1_ref, w2_ref, h_ref):
    x = x_ref[...]
    s = s_ref[...]
    h = (jnp.dot(x + s, w1_ref[...], preferred_element_type=jnp.float32)
         + jnp.dot(x * s, w2_ref[...], preferred_element_type=jnp.float32))
    h = jnp.where(h > 0, h, 0.2 * h)
    n = jnp.sqrt(jnp.sum(h * h, axis=1, keepdims=True))
    h_ref[...] = h / jnp.maximum(n, 1e-12)


_dense = pl.pallas_call(
    _dense_body,
    grid=(NNODES // BLK,),
    in_specs=[
        pl.BlockSpec((BLK, K), lambda i: (i, 0)),
        pl.BlockSpec((BLK, K), lambda i: (i, 0)),
        pl.BlockSpec((K, K), lambda i: (0, 0)),
        pl.BlockSpec((K, K), lambda i: (0, 0)),
    ],
    out_specs=pl.BlockSpec((BLK, K), lambda i: (i, 0)),
    out_shape=jax.ShapeDtypeStruct((NNODES, K), jnp.float32),
)

BBLK = 2048


def _xui_body(gu_ref, gi_ref, o_ref):
    o_ref[...] = jnp.sum(gu_ref[...] * gi_ref[...], axis=(0, 2))


_xui = pl.pallas_call(
    _xui_body,
    grid=(BATCH // BBLK,),
    in_specs=[
        pl.BlockSpec((4, BBLK, K), lambda i: (0, i, 0)),
        pl.BlockSpec((4, BBLK, K), lambda i: (0, i, 0)),
    ],
    out_specs=pl.BlockSpec((BBLK,), lambda i: (i,)),
    out_shape=jax.ShapeDtypeStruct((BATCH,), jnp.float32),
)


def kernel(Gu, Gi, edge_index, users, items,
           W1_0, W2_0, W1_1, W2_1, W1_2, W2_2):
    e0 = edge_index[0].astype(jnp.int32)
    e1 = edge_index[1].astype(jnp.int32)
    src2 = jnp.stack([e1 + NU, e0])
    dst2 = jnp.stack([e0, e1])
    zeros_pad = jnp.zeros((RPW, K), jnp.float32)

    x = jnp.concatenate([Gu, Gi], axis=0)
    tabs = [x]
    for (W1, W2) in ((W1_0, W2_0), (W1_1, W2_1), (W1_2, W2_2)):
        s = _spmv(src2, dst2, x, zeros_pad)
        x = _dense(x, s, W1, W2)
        tabs.append(x)

    uid = users.astype(jnp.int32)
    iid = items.astype(jnp.int32) + NU
    gu4, gi4 = _gather(tabs[0], tabs[1], tabs[2], tabs[3], uid, iid)

    xui = _xui(gu4, gi4)
    gamma_u = gu4.transpose(1, 0, 2).reshape(BATCH, 4 * K)
    gamma_i = gi4.transpose(1, 0, 2).reshape(BATCH, 4 * K)
    return (xui, gamma_u, gamma_i)


# speedup vs baseline: 13.7332x; 13.7332x over previous
import functools

import jax
import jax.numpy as jnp
from jax import lax
from jax.experimental import pallas as pl
from jax.experimental.pallas import tpu as pltpu
from jax.experimental.pallas import tpu_sc as plsc

NU = 25000
NI = 25000
NNODES = NU + NI
K = 64
KH = 32
NE = 400000
BATCH = 16384

NC = 2
NS = 16
EPW = NE // NS
CH = 1000
NCH = EPW // CH
HALF = NU
RPW = 1568
SP_ROWS = RPW * NS
TAIL = HALF - (NS - 1) * RPW

BPW = BATCH // (NC * NS)

_MESH = plsc.VectorSubcoreMesh(core_axis_name="c", subcore_axis_name="s")
_SC_PARAMS = pltpu.CompilerParams(use_tc_tiling_on_sc=False)


def _spmv_body(src_hbm, dst_hbm, xh_hbm, zero_hbm, out_hbm,
               sidx_v, didx_v, rows_v, s_sh, sem):
    c = lax.axis_index("c")
    t = lax.axis_index("s")

    pltpu.sync_copy(zero_hbm, s_sh.at[pl.ds(t * RPW, RPW)])
    plsc.subcore_barrier()

    ebase = c * NE + t * EPW

    def chunk(g, carry):
        off = ebase + g * CH
        pltpu.sync_copy(src_hbm.at[pl.ds(off, CH)], sidx_v)
        pltpu.sync_copy(dst_hbm.at[pl.ds(off, CH)], didx_v)
        pltpu.async_copy(xh_hbm.at[sidx_v], rows_v, sem).wait()
        pltpu.sync_copy(rows_v, s_sh.at[didx_v], add=True)
        return carry

    lax.fori_loop(0, NCH, chunk, 0)
    plsc.subcore_barrier()

    rbase = t * RPW

    @pl.when(t < NS - 1)
    def _():
        pltpu.sync_copy(s_sh.at[pl.ds(rbase, RPW)],
                        out_hbm.at[pl.ds(c * HALF + rbase, RPW)])

    @pl.when(t == NS - 1)
    def _():
        pltpu.sync_copy(s_sh.at[pl.ds(rbase, TAIL)],
                        out_hbm.at[pl.ds(c * HALF + rbase, TAIL)])


_spmv = functools.partial(
    pl.kernel,
    out_type=jax.ShapeDtypeStruct((NNODES, KH), jnp.float32),
    mesh=_MESH,
    compiler_params=_SC_PARAMS,
    scratch_types=[
        pltpu.VMEM((CH,), jnp.int32),
        pltpu.VMEM((CH,), jnp.int32),
        pltpu.VMEM((CH, KH), jnp.float32),
        pltpu.VMEM_SHARED((SP_ROWS, KH), jnp.float32),
        pltpu.SemaphoreType.DMA,
    ],
)(_spmv_body)


def _gather_body(t0, t1, t2, t3, t4, t5, t6, t7, uid_hbm, iid_hbm,
                 gu_hbm, gi_hbm, idx_v, rows_v, sem):
    c = lax.axis_index("c")
    t = lax.axis_index("s")
    wid = t * NC + c
    base = wid * BPW

    for ids, out in ((uid_hbm, gu_hbm), (iid_hbm, gi_hbm)):
        pltpu.sync_copy(ids.at[pl.ds(base, BPW)], idx_v)
        for l, tab in enumerate((t0, t1, t2, t3, t4, t5, t6, t7)):
            pltpu.async_copy(tab.at[idx_v], rows_v, sem).wait()
            pltpu.sync_copy(rows_v, out.at[l, pl.ds(base, BPW)])


_gather = functools.partial(
    pl.kernel,
    out_type=(jax.ShapeDtypeStruct((8, BATCH, KH), jnp.float32),
              jax.ShapeDtypeStruct((8, BATCH, KH), jnp.float32)),
    mesh=_MESH,
    compiler_params=_SC_PARAMS,
    scratch_types=[
        pltpu.VMEM((BPW,), jnp.int32),
        pltpu.VMEM((BPW, KH), jnp.float32),
        pltpu.SemaphoreType.DMA,
    ],
)(_gather_body)


BLK = 2000


def _dense_body(xa_ref, xb_ref, sa_ref, sb_ref, w1_ref, w2_ref,
                ha_ref, hb_ref):
    x = jnp.concatenate([xa_ref[...], xb_ref[...]], axis=1)
    s = jnp.concatenate([sa_ref[...], sb_ref[...]], axis=1)
    h = (jnp.dot(x + s, w1_ref[...], preferred_element_type=jnp.float32)
         + jnp.dot(x * s, w2_ref[...], preferred_element_type=jnp.float32))
    h = jnp.where(h > 0, h, 0.2 * h)
    n = jnp.sqrt(jnp.sum(h * h, axis=1, keepdims=True))
    h = h / jnp.maximum(n, 1e-12)
    ha_ref[...] = h[:, :KH]
    hb_ref[...] = h[:, KH:]


_half_spec = pl.BlockSpec((BLK, KH), lambda i: (i, 0))
_dense = pl.pallas_call(
    _dense_body,
    grid=(NNODES // BLK,),
    in_specs=[
        _half_spec, _half_spec, _half_spec, _half_spec,
        pl.BlockSpec((K, K), lambda i: (0, 0)),
        pl.BlockSpec((K, K), lambda i: (0, 0)),
    ],
    out_specs=(_half_spec, _half_spec),
    out_shape=(jax.ShapeDtypeStruct((NNODES, KH), jnp.float32),
               jax.ShapeDtypeStruct((NNODES, KH), jnp.float32)),
)

BBLK = 2048


def _xui_body(gu_ref, gi_ref, o_ref):
    o_ref[...] = jnp.sum(gu_ref[...] * gi_ref[...], axis=(0, 2))


_xui = pl.pallas_call(
    _xui_body,
    grid=(BATCH // BBLK,),
    in_specs=[
        pl.BlockSpec((8, BBLK, KH), lambda i: (0, i, 0)),
        pl.BlockSpec((8, BBLK, KH), lambda i: (0, i, 0)),
    ],
    out_specs=pl.BlockSpec((BBLK,), lambda i: (i,)),
    out_shape=jax.ShapeDtypeStruct((BATCH,), jnp.float32),
)


def kernel(Gu, Gi, edge_index, users, items,
           W1_0, W2_0, W1_1, W2_1, W1_2, W2_2):
    e0 = edge_index[0].astype(jnp.int32)
    e1 = edge_index[1].astype(jnp.int32)
    src2 = jnp.concatenate([e1 + NU, e0])
    dst2 = jnp.concatenate([e0, e1])
    zeros_pad = jnp.zeros((RPW, KH), jnp.float32)

    x0 = jnp.concatenate([Gu, Gi], axis=0)
    xa, xb = x0[:, :KH], x0[:, KH:]
    tabs = [xa, xb]
    for (W1, W2) in ((W1_0, W2_0), (W1_1, W2_1), (W1_2, W2_2)):
        sa = _spmv(src2, dst2, xa, zeros_pad)
        sb = _spmv(src2, dst2, xb, zeros_pad)
        xa, xb = _dense(xa, xb, sa, sb, W1, W2)
        tabs += [xa, xb]

    uid = users.astype(jnp.int32)
    iid = items.astype(jnp.int32) + NU
    gu8, gi8 = _gather(*tabs, uid, iid)

    xui = _xui(gu8, gi8)
    gamma_u = gu8.transpose(1, 0, 2).reshape(BATCH, 4 * K)
    gamma_i = gi8.transpose(1, 0, 2).reshape(BATCH, 4 * K)
    return (xui, gamma_u, gamma_i)


# speedup vs baseline: 15.8166x; 1.1517x over previous
import functools

import jax
import jax.numpy as jnp
from jax import lax
from jax.experimental import pallas as pl
from jax.experimental.pallas import tpu as pltpu
from jax.experimental.pallas import tpu_sc as plsc

NU = 25000
NI = 25000
NNODES = NU + NI
K = 64
KH = 32
NE = 400000
BATCH = 16384

NC = 2
NS = 16
EPW = NE // NS
CH = 1000
NCH = EPW // CH
HALF = NU
RPW = 1568
SP_ROWS = RPW * NS
TAIL = HALF - (NS - 1) * RPW

BPW = BATCH // (NC * NS)

_MESH = plsc.VectorSubcoreMesh(core_axis_name="c", subcore_axis_name="s")
_SC_PARAMS = pltpu.CompilerParams(use_tc_tiling_on_sc=False)


def _spmv_body(src_hbm, dst_hbm, xa_hbm, xb_hbm, zero_hbm, sa_hbm, sb_hbm,
               sidx2, didx2, rows2, s_sh, sem2):
    c = lax.axis_index("c")
    t = lax.axis_index("s")
    ebase = c * NE + t * EPW
    rbase = t * RPW

    for xh_hbm, out_hbm in ((xa_hbm, sa_hbm), (xb_hbm, sb_hbm)):
        pltpu.sync_copy(zero_hbm, s_sh.at[pl.ds(rbase, RPW)])
        plsc.subcore_barrier()

        pltpu.sync_copy(src_hbm.at[pl.ds(ebase, CH)], sidx2.at[0])
        pltpu.sync_copy(dst_hbm.at[pl.ds(ebase, CH)], didx2.at[0])
        pltpu.async_copy(xh_hbm.at[sidx2.at[0]], rows2.at[0], sem2.at[0])

        def chunk(g, carry):
            p = lax.rem(g, 2)
            pn = 1 - p

            @pl.when(g + 1 < NCH)
            def _():
                off = ebase + (g + 1) * CH
                pltpu.sync_copy(src_hbm.at[pl.ds(off, CH)], sidx2.at[pn])
                pltpu.sync_copy(dst_hbm.at[pl.ds(off, CH)], didx2.at[pn])
                pltpu.async_copy(xh_hbm.at[sidx2.at[pn]], rows2.at[pn],
                                 sem2.at[pn])

            pltpu.make_async_copy(xh_hbm.at[sidx2.at[p]], rows2.at[p],
                                  sem2.at[p]).wait()
            pltpu.sync_copy(rows2.at[p], s_sh.at[didx2.at[p]], add=True)
            return carry

        lax.fori_loop(0, NCH, chunk, 0)
        plsc.subcore_barrier()

        @pl.when(t < NS - 1)
        def _():
            pltpu.sync_copy(s_sh.at[pl.ds(rbase, RPW)],
                            out_hbm.at[pl.ds(c * HALF + rbase, RPW)])

        @pl.when(t == NS - 1)
        def _():
            pltpu.sync_copy(s_sh.at[pl.ds(rbase, TAIL)],
                            out_hbm.at[pl.ds(c * HALF + rbase, TAIL)])


_spmv = functools.partial(
    pl.kernel,
    out_type=(jax.ShapeDtypeStruct((NNODES, KH), jnp.float32),
              jax.ShapeDtypeStruct((NNODES, KH), jnp.float32)),
    mesh=_MESH,
    compiler_params=_SC_PARAMS,
    scratch_types=[
        pltpu.VMEM((2, CH), jnp.int32),
        pltpu.VMEM((2, CH), jnp.int32),
        pltpu.VMEM((2, CH, KH), jnp.float32),
        pltpu.VMEM_SHARED((SP_ROWS, KH), jnp.float32),
        pltpu.SemaphoreType.DMA((2,)),
    ],
)(_spmv_body)


def _gather_body(t0, t1, t2, t3, t4, t5, t6, t7, uid_hbm, iid_hbm,
                 gu_hbm, gi_hbm, idx2, rows2, sem2):
    c = lax.axis_index("c")
    t = lax.axis_index("s")
    wid = t * NC + c
    base = wid * BPW

    pltpu.sync_copy(uid_hbm.at[pl.ds(base, BPW)], idx2.at[0])
    pltpu.sync_copy(iid_hbm.at[pl.ds(base, BPW)], idx2.at[1])

    tabs = (t0, t1, t2, t3, t4, t5, t6, t7)
    outs = (gu_hbm, gi_hbm)
    ops = [(s, l) for s in (0, 1) for l in range(len(tabs))]
    prev = None
    for k, (s, l) in enumerate(ops):
        p = k % 2
        pltpu.async_copy(tabs[l].at[idx2.at[s]], rows2.at[p], sem2.at[p])
        if prev is not None:
            qs, ql, qp = prev
            pltpu.make_async_copy(tabs[ql].at[idx2.at[qs]], rows2.at[qp],
                                  sem2.at[qp]).wait()
            pltpu.sync_copy(rows2.at[qp], outs[qs].at[ql, pl.ds(base, BPW)])
        prev = (s, l, p)
    qs, ql, qp = prev
    pltpu.make_async_copy(tabs[ql].at[idx2.at[qs]], rows2.at[qp],
                          sem2.at[qp]).wait()
    pltpu.sync_copy(rows2.at[qp], outs[qs].at[ql, pl.ds(base, BPW)])


_gather = functools.partial(
    pl.kernel,
    out_type=(jax.ShapeDtypeStruct((8, BATCH, KH), jnp.float32),
              jax.ShapeDtypeStruct((8, BATCH, KH), jnp.float32)),
    mesh=_MESH,
    compiler_params=_SC_PARAMS,
    scratch_types=[
        pltpu.VMEM((2, BPW), jnp.int32),
        pltpu.VMEM((2, BPW, KH), jnp.float32),
        pltpu.SemaphoreType.DMA((2,)),
    ],
)(_gather_body)


BLK = 2000


def _dense_body(xa_ref, xb_ref, sa_ref, sb_ref, w1_ref, w2_ref,
                ha_ref, hb_ref):
    x = jnp.concatenate([xa_ref[...], xb_ref[...]], axis=1)
    s = jnp.concatenate([sa_ref[...], sb_ref[...]], axis=1)
    h = (jnp.dot(x + s, w1_ref[...], preferred_element_type=jnp.float32)
         + jnp.dot(x * s, w2_ref[...], preferred_element_type=jnp.float32))
    h = jnp.where(h > 0, h, 0.2 * h)
    n = jnp.sqrt(jnp.sum(h * h, axis=1, keepdims=True))
    h = h / jnp.maximum(n, 1e-12)
    ha_ref[...] = h[:, :KH]
    hb_ref[...] = h[:, KH:]


_half_spec = pl.BlockSpec((BLK, KH), lambda i: (i, 0))
_dense = pl.pallas_call(
    _dense_body,
    grid=(NNODES // BLK,),
    in_specs=[
        _half_spec, _half_spec, _half_spec, _half_spec,
        pl.BlockSpec((K, K), lambda i: (0, 0)),
        pl.BlockSpec((K, K), lambda i: (0, 0)),
    ],
    out_specs=(_half_spec, _half_spec),
    out_shape=(jax.ShapeDtypeStruct((NNODES, KH), jnp.float32),
               jax.ShapeDtypeStruct((NNODES, KH), jnp.float32)),
)

BBLK = 2048


def _xui_body(gu_ref, gi_ref, o_ref):
    o_ref[...] = jnp.sum(gu_ref[...] * gi_ref[...], axis=(0, 2))


_xui = pl.pallas_call(
    _xui_body,
    grid=(BATCH // BBLK,),
    in_specs=[
        pl.BlockSpec((8, BBLK, KH), lambda i: (0, i, 0)),
        pl.BlockSpec((8, BBLK, KH), lambda i: (0, i, 0)),
    ],
    out_specs=pl.BlockSpec((BBLK,), lambda i: (i,)),
    out_shape=jax.ShapeDtypeStruct((BATCH,), jnp.float32),
)


def kernel(Gu, Gi, edge_index, users, items,
           W1_0, W2_0, W1_1, W2_1, W1_2, W2_2):
    e0 = edge_index[0].astype(jnp.int32)
    e1 = edge_index[1].astype(jnp.int32)
    src2 = jnp.concatenate([e1 + NU, e0])
    dst2 = jnp.concatenate([e0, e1])
    zeros_pad = jnp.zeros((RPW, KH), jnp.float32)

    x0 = jnp.concatenate([Gu, Gi], axis=0)
    xa, xb = x0[:, :KH], x0[:, KH:]
    tabs = [xa, xb]
    for (W1, W2) in ((W1_0, W2_0), (W1_1, W2_1), (W1_2, W2_2)):
        sa, sb = _spmv(src2, dst2, xa, xb, zeros_pad)
        xa, xb = _dense(xa, xb, sa, sb, W1, W2)
        tabs += [xa, xb]

    uid = users.astype(jnp.int32)
    iid = items.astype(jnp.int32) + NU
    gu8, gi8 = _gather(*tabs, uid, iid)

    xui = _xui(gu8, gi8)
    gamma_u = gu8.transpose(1, 0, 2).reshape(BATCH, 4 * K)
    gamma_i = gi8.transpose(1, 0, 2).reshape(BATCH, 4 * K)
    return (xui, gamma_u, gamma_i)


# speedup vs baseline: 16.9577x; 1.0721x over previous
import functools

import jax
import jax.numpy as jnp
from jax import lax
from jax.experimental import pallas as pl
from jax.experimental.pallas import tpu as pltpu
from jax.experimental.pallas import tpu_sc as plsc

NU = 25000
NI = 25000
NNODES = NU + NI
K = 64
KH = 32
NE = 400000
BATCH = 16384

NC = 2
NS = 16
EPW = NE // NS
CH = 1000
NCH = EPW // CH
HALF = NU
RPW = 1568
SP_ROWS = RPW * NS
TAIL = HALF - (NS - 1) * RPW

BPW = BATCH // (NC * NS)

_MESH = plsc.VectorSubcoreMesh(core_axis_name="c", subcore_axis_name="s")
_SC_PARAMS = pltpu.CompilerParams(use_tc_tiling_on_sc=False)


def _spmv_body(src_hbm, dst_hbm, xa_hbm, xb_hbm, zero_hbm, sa_hbm, sb_hbm,
               sidx, didx, rows2, s_sh, gsem2, ssem2, isem3):
    c = lax.axis_index("c")
    t = lax.axis_index("s")
    rbase = t * RPW
    ebase = c * NE + t * EPW

    def idx_load(g, slot):
        pltpu.async_copy(src_hbm.at[pl.ds(ebase + g * CH, CH)],
                         sidx.at[slot], isem3.at[slot])
        pltpu.async_copy(dst_hbm.at[pl.ds(ebase + g * CH, CH)],
                         didx.at[slot], isem3.at[slot])

    def idx_wait(g, slot):
        pltpu.make_async_copy(src_hbm.at[pl.ds(ebase + g * CH, CH)],
                              sidx.at[slot], isem3.at[slot]).wait()
        pltpu.make_async_copy(dst_hbm.at[pl.ds(ebase + g * CH, CH)],
                              didx.at[slot], isem3.at[slot]).wait()

    for xh_hbm, out_hbm in ((xa_hbm, sa_hbm), (xb_hbm, sb_hbm)):
        pltpu.sync_copy(zero_hbm, s_sh.at[pl.ds(rbase, RPW)])
        plsc.subcore_barrier()

        idx_load(0, 0)
        idx_load(1, 1)
        idx_wait(0, 0)
        pltpu.async_copy(xh_hbm.at[sidx.at[0]], rows2.at[0], gsem2.at[0])

        def chunk(g, carry):
            p = lax.rem(g, 2)
            pn = 1 - p
            r = lax.rem(g, 3)

            @pl.when(g >= 1)
            def _():
                pltpu.make_async_copy(rows2.at[pn],
                                      s_sh.at[didx.at[lax.rem(g + 2, 3)]],
                                      ssem2.at[pn]).wait()

            @pl.when(g + 1 < NCH)
            def _():
                rn = lax.rem(g + 1, 3)
                idx_wait(g + 1, rn)
                pltpu.async_copy(xh_hbm.at[sidx.at[rn]], rows2.at[pn],
                                 gsem2.at[pn])

                @pl.when(g + 2 < NCH)
                def _():
                    idx_load(g + 2, lax.rem(g + 2, 3))

            pltpu.make_async_copy(xh_hbm.at[sidx.at[r]], rows2.at[p],
                                  gsem2.at[p]).wait()
            pltpu.async_copy(rows2.at[p], s_sh.at[didx.at[r]], ssem2.at[p],
                             add=True)
            return carry

        lax.fori_loop(0, NCH, chunk, 0)
        pltpu.make_async_copy(rows2.at[(NCH - 1) % 2],
                              s_sh.at[didx.at[(NCH - 1) % 3]],
                              ssem2.at[(NCH - 1) % 2]).wait()
        plsc.subcore_barrier()

        @pl.when(t < NS - 1)
        def _():
            pltpu.sync_copy(s_sh.at[pl.ds(rbase, RPW)],
                            out_hbm.at[pl.ds(c * HALF + rbase, RPW)])

        @pl.when(t == NS - 1)
        def _():
            pltpu.sync_copy(s_sh.at[pl.ds(rbase, TAIL)],
                            out_hbm.at[pl.ds(c * HALF + rbase, TAIL)])


_spmv = functools.partial(
    pl.kernel,
    out_type=(jax.ShapeDtypeStruct((NNODES, KH), jnp.float32),
              jax.ShapeDtypeStruct((NNODES, KH), jnp.float32)),
    mesh=_MESH,
    compiler_params=_SC_PARAMS,
    scratch_types=[
        pltpu.VMEM((3, CH), jnp.int32),
        pltpu.VMEM((3, CH), jnp.int32),
        pltpu.VMEM((2, CH, KH), jnp.float32),
        pltpu.VMEM_SHARED((SP_ROWS, KH), jnp.float32),
        pltpu.SemaphoreType.DMA((2,)),
        pltpu.SemaphoreType.DMA((2,)),
        pltpu.SemaphoreType.DMA((3,)),
    ],
)(_spmv_body)


def _gather_body(t0, t1, t2, t3, t4, t5, t6, t7, uid_hbm, iid_hbm,
                 gu_hbm, gi_hbm, idx2, rows2, sem2):
    c = lax.axis_index("c")
    t = lax.axis_index("s")
    wid = t * NC + c
    base = wid * BPW

    pltpu.sync_copy(uid_hbm.at[pl.ds(base, BPW)], idx2.at[0])
    pltpu.sync_copy(iid_hbm.at[pl.ds(base, BPW)], idx2.at[1])

    tabs = (t0, t1, t2, t3, t4, t5, t6, t7)
    outs = (gu_hbm, gi_hbm)
    ops = [(s, l) for s in (0, 1) for l in range(len(tabs))]
    prev = None
    for k, (s, l) in enumerate(ops):
        p = k % 2
        pltpu.async_copy(tabs[l].at[idx2.at[s]], rows2.at[p], sem2.at[p])
        if prev is not None:
            qs, ql, qp = prev
            pltpu.make_async_copy(tabs[ql].at[idx2.at[qs]], rows2.at[qp],
                                  sem2.at[qp]).wait()
            pltpu.sync_copy(rows2.at[qp], outs[qs].at[ql, pl.ds(base, BPW)])
        prev = (s, l, p)
    qs, ql, qp = prev
    pltpu.make_async_copy(tabs[ql].at[idx2.at[qs]], rows2.at[qp],
                          sem2.at[qp]).wait()
    pltpu.sync_copy(rows2.at[qp], outs[qs].at[ql, pl.ds(base, BPW)])


_gather = functools.partial(
    pl.kernel,
    out_type=(jax.ShapeDtypeStruct((8, BATCH, KH), jnp.float32),
              jax.ShapeDtypeStruct((8, BATCH, KH), jnp.float32)),
    mesh=_MESH,
    compiler_params=_SC_PARAMS,
    scratch_types=[
        pltpu.VMEM((2, BPW), jnp.int32),
        pltpu.VMEM((2, BPW, KH), jnp.float32),
        pltpu.SemaphoreType.DMA((2,)),
    ],
)(_gather_body)


BLK = 2000


def _dense_body(xa_ref, xb_ref, sa_ref, sb_ref, w1_ref, w2_ref,
                ha_ref, hb_ref):
    x = jnp.concatenate([xa_ref[...], xb_ref[...]], axis=1)
    s = jnp.concatenate([sa_ref[...], sb_ref[...]], axis=1)
    h = (jnp.dot(x + s, w1_ref[...], preferred_element_type=jnp.float32)
         + jnp.dot(x * s, w2_ref[...], preferred_element_type=jnp.float32))
    h = jnp.where(h > 0, h, 0.2 * h)
    n = jnp.sqrt(jnp.sum(h * h, axis=1, keepdims=True))
    h = h / jnp.maximum(n, 1e-12)
    ha_ref[...] = h[:, :KH]
    hb_ref[...] = h[:, KH:]


_half_spec = pl.BlockSpec((BLK, KH), lambda i: (i, 0))
_dense = pl.pallas_call(
    _dense_body,
    grid=(NNODES // BLK,),
    in_specs=[
        _half_spec, _half_spec, _half_spec, _half_spec,
        pl.BlockSpec((K, K), lambda i: (0, 0)),
        pl.BlockSpec((K, K), lambda i: (0, 0)),
    ],
    out_specs=(_half_spec, _half_spec),
    out_shape=(jax.ShapeDtypeStruct((NNODES, KH), jnp.float32),
               jax.ShapeDtypeStruct((NNODES, KH), jnp.float32)),
)

BBLK = 2048


def _xui_body(gu_ref, gi_ref, o_ref):
    o_ref[...] = jnp.sum(gu_ref[...] * gi_ref[...], axis=(0, 2))


_xui = pl.pallas_call(
    _xui_body,
    grid=(BATCH // BBLK,),
    in_specs=[
        pl.BlockSpec((8, BBLK, KH), lambda i: (0, i, 0)),
        pl.BlockSpec((8, BBLK, KH), lambda i: (0, i, 0)),
    ],
    out_specs=pl.BlockSpec((BBLK,), lambda i: (i,)),
    out_shape=jax.ShapeDtypeStruct((BATCH,), jnp.float32),
)


def kernel(Gu, Gi, edge_index, users, items,
           W1_0, W2_0, W1_1, W2_1, W1_2, W2_2):
    e0 = edge_index[0].astype(jnp.int32)
    e1 = edge_index[1].astype(jnp.int32)
    src2 = jnp.concatenate([e1 + NU, e0])
    dst2 = jnp.concatenate([e0, e1])
    zeros_pad = jnp.zeros((RPW, KH), jnp.float32)

    x0 = jnp.concatenate([Gu, Gi], axis=0)
    xa, xb = x0[:, :KH], x0[:, KH:]
    tabs = [xa, xb]
    for (W1, W2) in ((W1_0, W2_0), (W1_1, W2_1), (W1_2, W2_2)):
        sa, sb = _spmv(src2, dst2, xa, xb, zeros_pad)
        xa, xb = _dense(xa, xb, sa, sb, W1, W2)
        tabs += [xa, xb]

    uid = users.astype(jnp.int32)
    iid = items.astype(jnp.int32) + NU
    gu8, gi8 = _gather(*tabs, uid, iid)

    xui = _xui(gu8, gi8)
    gamma_u = gu8.transpose(1, 0, 2).reshape(BATCH, 4 * K)
    gamma_i = gi8.transpose(1, 0, 2).reshape(BATCH, 4 * K)
    return (xui, gamma_u, gamma_i)


# speedup vs baseline: 17.1479x; 1.0112x over previous
import functools

import jax
import jax.numpy as jnp
from jax import lax
from jax.experimental import pallas as pl
from jax.experimental.pallas import tpu as pltpu
from jax.experimental.pallas import tpu_sc as plsc

NU = 25000
NI = 25000
NNODES = NU + NI
K = 64
KH = 32
NE = 400000
BATCH = 16384

NC = 2
NS = 16
EPW = NE // NS
CH = 1000
NCH = EPW // CH
HALF = NU
RPW = 1568
SP_ROWS = RPW * NS
TAIL = HALF - (NS - 1) * RPW

BPW = BATCH // (NC * NS)

_MESH = plsc.VectorSubcoreMesh(core_axis_name="c", subcore_axis_name="s")
_SC_PARAMS = pltpu.CompilerParams(use_tc_tiling_on_sc=False)


def _spmv_body(src_hbm, dst_hbm, xa_hbm, xb_hbm, zero_hbm, sa_hbm, sb_hbm,
               sidx, didx, rows2, s_sh, gsem2, ssem2, isem3):
    c = lax.axis_index("c")
    t = lax.axis_index("s")
    rbase = t * RPW
    ebase = c * NE + t * EPW

    def idx_load(g, slot):
        pltpu.async_copy(src_hbm.at[pl.ds(ebase + g * CH, CH)],
                         sidx.at[slot], isem3.at[slot])
        pltpu.async_copy(dst_hbm.at[pl.ds(ebase + g * CH, CH)],
                         didx.at[slot], isem3.at[slot])

    def idx_wait(g, slot):
        pltpu.make_async_copy(src_hbm.at[pl.ds(ebase + g * CH, CH)],
                              sidx.at[slot], isem3.at[slot]).wait()
        pltpu.make_async_copy(dst_hbm.at[pl.ds(ebase + g * CH, CH)],
                              didx.at[slot], isem3.at[slot]).wait()

    for xh_hbm, out_hbm in ((xa_hbm, sa_hbm), (xb_hbm, sb_hbm)):
        pltpu.sync_copy(zero_hbm, s_sh.at[pl.ds(rbase, RPW)])
        plsc.subcore_barrier()

        idx_load(0, 0)
        idx_load(1, 1)
        idx_wait(0, 0)
        pltpu.async_copy(xh_hbm.at[sidx.at[0]], rows2.at[0], gsem2.at[0])

        def chunk(g, carry):
            p = lax.rem(g, 2)
            pn = 1 - p
            r = lax.rem(g, 3)

            @pl.when(g >= 1)
            def _():
                pltpu.make_async_copy(rows2.at[pn],
                                      s_sh.at[didx.at[lax.rem(g + 2, 3)]],
                                      ssem2.at[pn]).wait()

            @pl.when(g + 1 < NCH)
            def _():
                rn = lax.rem(g + 1, 3)
                idx_wait(g + 1, rn)
                pltpu.async_copy(xh_hbm.at[sidx.at[rn]], rows2.at[pn],
                                 gsem2.at[pn])

                @pl.when(g + 2 < NCH)
                def _():
                    idx_load(g + 2, lax.rem(g + 2, 3))

            pltpu.make_async_copy(xh_hbm.at[sidx.at[r]], rows2.at[p],
                                  gsem2.at[p]).wait()
            pltpu.async_copy(rows2.at[p], s_sh.at[didx.at[r]], ssem2.at[p],
                             add=True)
            return carry

        lax.fori_loop(0, NCH, chunk, 0)
        pltpu.make_async_copy(rows2.at[(NCH - 1) % 2],
                              s_sh.at[didx.at[(NCH - 1) % 3]],
                              ssem2.at[(NCH - 1) % 2]).wait()
        plsc.subcore_barrier()

        @pl.when(t < NS - 1)
        def _():
            pltpu.sync_copy(s_sh.at[pl.ds(rbase, RPW)],
                            out_hbm.at[pl.ds(c * HALF + rbase, RPW)])

        @pl.when(t == NS - 1)
        def _():
            pltpu.sync_copy(s_sh.at[pl.ds(rbase, TAIL)],
                            out_hbm.at[pl.ds(c * HALF + rbase, TAIL)])


_spmv = functools.partial(
    pl.kernel,
    out_type=(jax.ShapeDtypeStruct((NNODES, KH), jnp.float32),
              jax.ShapeDtypeStruct((NNODES, KH), jnp.float32)),
    mesh=_MESH,
    compiler_params=_SC_PARAMS,
    scratch_types=[
        pltpu.VMEM((3, CH), jnp.int32),
        pltpu.VMEM((3, CH), jnp.int32),
        pltpu.VMEM((2, CH, KH), jnp.float32),
        pltpu.VMEM_SHARED((SP_ROWS, KH), jnp.float32),
        pltpu.SemaphoreType.DMA((2,)),
        pltpu.SemaphoreType.DMA((2,)),
        pltpu.SemaphoreType.DMA((3,)),
    ],
)(_spmv_body)


def _gather_body(t0, t1, t2, t3, t4, t5, t6, t7, uid_hbm, iid_hbm,
                 gu_hbm, gi_hbm, idx2, rows2, sem2):
    c = lax.axis_index("c")
    t = lax.axis_index("s")
    wid = t * NC + c
    base = wid * BPW

    pltpu.sync_copy(uid_hbm.at[pl.ds(base, BPW)], idx2.at[0])
    pltpu.sync_copy(iid_hbm.at[pl.ds(base, BPW)], idx2.at[1])

    tabs = (t0, t1, t2, t3, t4, t5, t6, t7)
    outs = (gu_hbm, gi_hbm)
    ops = [(s, l) for s in (0, 1) for l in range(len(tabs))]
    prev = None
    for k, (s, l) in enumerate(ops):
        p = k % 2
        pltpu.async_copy(tabs[l].at[idx2.at[s]], rows2.at[p], sem2.at[p])
        if prev is not None:
            qs, ql, qp = prev
            pltpu.make_async_copy(tabs[ql].at[idx2.at[qs]], rows2.at[qp],
                                  sem2.at[qp]).wait()
            pltpu.sync_copy(rows2.at[qp], outs[qs].at[ql, pl.ds(base, BPW)])
        prev = (s, l, p)
    qs, ql, qp = prev
    pltpu.make_async_copy(tabs[ql].at[idx2.at[qs]], rows2.at[qp],
                          sem2.at[qp]).wait()
    pltpu.sync_copy(rows2.at[qp], outs[qs].at[ql, pl.ds(base, BPW)])


_gather = functools.partial(
    pl.kernel,
    out_type=(jax.ShapeDtypeStruct((8, BATCH, KH), jnp.float32),
              jax.ShapeDtypeStruct((8, BATCH, KH), jnp.float32)),
    mesh=_MESH,
    compiler_params=_SC_PARAMS,
    scratch_types=[
        pltpu.VMEM((2, BPW), jnp.int32),
        pltpu.VMEM((2, BPW, KH), jnp.float32),
        pltpu.SemaphoreType.DMA((2,)),
    ],
)(_gather_body)


BLK = 2000


def _dense_body(xa_ref, xb_ref, sa_ref, sb_ref, w1_ref, w2_ref,
                ha_ref, hb_ref):
    x = jnp.concatenate([xa_ref[...], xb_ref[...]], axis=1)
    s = jnp.concatenate([sa_ref[...], sb_ref[...]], axis=1)
    h = (jnp.dot(x + s, w1_ref[...], preferred_element_type=jnp.float32)
         + jnp.dot(x * s, w2_ref[...], preferred_element_type=jnp.float32))
    h = jnp.where(h > 0, h, 0.2 * h)
    n = jnp.sqrt(jnp.sum(h * h, axis=1, keepdims=True))
    h = h / jnp.maximum(n, 1e-12)
    ha_ref[...] = h[:, :KH]
    hb_ref[...] = h[:, KH:]


_half_spec = pl.BlockSpec((BLK, KH), lambda i: (i, 0))
_dense = pl.pallas_call(
    _dense_body,
    grid=(NNODES // BLK,),
    in_specs=[
        _half_spec, _half_spec, _half_spec, _half_spec,
        pl.BlockSpec((K, K), lambda i: (0, 0)),
        pl.BlockSpec((K, K), lambda i: (0, 0)),
    ],
    out_specs=(_half_spec, _half_spec),
    out_shape=(jax.ShapeDtypeStruct((NNODES, KH), jnp.float32),
               jax.ShapeDtypeStruct((NNODES, KH), jnp.float32)),
)

BBLK = 2048


def _finale_body(gu_ref, gi_ref, o_ref, gau_ref, gai_ref):
    gu = jnp.concatenate([gu_ref[l] for l in range(8)], axis=1)
    gi = jnp.concatenate([gi_ref[l] for l in range(8)], axis=1)
    o_ref[...] = jnp.sum(gu * gi, axis=1)
    gau_ref[...] = gu
    gai_ref[...] = gi


_finale = pl.pallas_call(
    _finale_body,
    grid=(BATCH // BBLK,),
    in_specs=[
        pl.BlockSpec((8, BBLK, KH), lambda i: (0, i, 0)),
        pl.BlockSpec((8, BBLK, KH), lambda i: (0, i, 0)),
    ],
    out_specs=(pl.BlockSpec((BBLK,), lambda i: (i,)),
               pl.BlockSpec((BBLK, 8 * KH), lambda i: (i, 0)),
               pl.BlockSpec((BBLK, 8 * KH), lambda i: (i, 0))),
    out_shape=(jax.ShapeDtypeStruct((BATCH,), jnp.float32),
               jax.ShapeDtypeStruct((BATCH, 8 * KH), jnp.float32),
               jax.ShapeDtypeStruct((BATCH, 8 * KH), jnp.float32)),
)

SBLK = 1000


def _split_body(gu_ref, gi_ref, xa_ref, xb_ref):
    i = pl.program_id(0)

    @pl.when(i < NU // SBLK)
    def _():
        xa_ref[...] = gu_ref[:, :KH]
        xb_ref[...] = gu_ref[:, KH:]

    @pl.when(i >= NU // SBLK)
    def _():
        xa_ref[...] = gi_ref[:, :KH]
        xb_ref[...] = gi_ref[:, KH:]


_split = pl.pallas_call(
    _split_body,
    grid=(NNODES // SBLK,),
    in_specs=[
        pl.BlockSpec((SBLK, K), lambda i: (jnp.minimum(i, NU // SBLK - 1), 0)),
        pl.BlockSpec((SBLK, K), lambda i: (jnp.maximum(i - NU // SBLK, 0), 0)),
    ],
    out_specs=(pl.BlockSpec((SBLK, KH), lambda i: (i, 0)),
               pl.BlockSpec((SBLK, KH), lambda i: (i, 0))),
    out_shape=(jax.ShapeDtypeStruct((NNODES, KH), jnp.float32),
               jax.ShapeDtypeStruct((NNODES, KH), jnp.float32)),
)


def kernel(Gu, Gi, edge_index, users, items,
           W1_0, W2_0, W1_1, W2_1, W1_2, W2_2):
    e0 = edge_index[0].astype(jnp.int32)
    e1 = edge_index[1].astype(jnp.int32)
    src2 = jnp.concatenate([e1 + NU, e0])
    dst2 = jnp.concatenate([e0, e1])
    zeros_pad = jnp.zeros((RPW, KH), jnp.float32)

    xa, xb = _split(Gu, Gi)
    tabs = [xa, xb]
    for (W1, W2) in ((W1_0, W2_0), (W1_1, W2_1), (W1_2, W2_2)):
        sa, sb = _spmv(src2, dst2, xa, xb, zeros_pad)
        xa, xb = _dense(xa, xb, sa, sb, W1, W2)
        tabs += [xa, xb]

    uid = users.astype(jnp.int32)
    iid = items.astype(jnp.int32) + NU
    gu8, gi8 = _gather(*tabs, uid, iid)
    xui, gamma_u, gamma_i = _finale(gu8, gi8)
    return (xui, gamma_u, gamma_i)


# speedup vs baseline: 24.4693x; 1.4270x over previous
import functools

import jax
import jax.numpy as jnp
from jax import lax
from jax.experimental import pallas as pl
from jax.experimental.pallas import tpu as pltpu
from jax.experimental.pallas import tpu_sc as plsc

NU = 25000
NI = 25000
K = 64
KH = 32
NE = 400000
BATCH = 16384

NC = 2
NS = 16
EPW = NE // NS
CH = 1000
NCH = EPW // CH
RPW = 1568
NPH = RPW * NS
NP = 2 * NPH
PR = NP * KH // 128

BPW = BATCH // (NC * NS)

_MESH = plsc.VectorSubcoreMesh(core_axis_name="c", subcore_axis_name="s")
_SC_PARAMS = pltpu.CompilerParams(use_tc_tiling_on_sc=False)


def _spmv_body(src_hbm, dst_hbm, xa_hbm, xb_hbm, zero_hbm, sa_hbm, sb_hbm,
               sidx, didx, rows2, s_sh, gsem2, ssem2, isem3):
    c = lax.axis_index("c")
    t = lax.axis_index("s")
    rbase = t * RPW
    ebase = c * NE + t * EPW

    def idx_load(g, slot):
        pltpu.async_copy(src_hbm.at[pl.ds(ebase + g * CH, CH)],
                         sidx.at[slot], isem3.at[slot])
        pltpu.async_copy(dst_hbm.at[pl.ds(ebase + g * CH, CH)],
                         didx.at[slot], isem3.at[slot])

    def idx_wait(g, slot):
        pltpu.make_async_copy(src_hbm.at[pl.ds(ebase + g * CH, CH)],
                              sidx.at[slot], isem3.at[slot]).wait()
        pltpu.make_async_copy(dst_hbm.at[pl.ds(ebase + g * CH, CH)],
                              didx.at[slot], isem3.at[slot]).wait()

    for xh_hbm, out_hbm in ((xa_hbm, sa_hbm), (xb_hbm, sb_hbm)):
        pltpu.sync_copy(zero_hbm, s_sh.at[pl.ds(rbase, RPW)])
        plsc.subcore_barrier()

        idx_load(0, 0)
        idx_load(1, 1)
        idx_wait(0, 0)
        pltpu.async_copy(xh_hbm.at[sidx.at[0]], rows2.at[0], gsem2.at[0])

        def chunk(g, carry):
            p = lax.rem(g, 2)
            pn = 1 - p
            r = lax.rem(g, 3)

            @pl.when(g >= 1)
            def _():
                pltpu.make_async_copy(rows2.at[pn],
                                      s_sh.at[didx.at[lax.rem(g + 2, 3)]],
                                      ssem2.at[pn]).wait()

            @pl.when(g + 1 < NCH)
            def _():
                rn = lax.rem(g + 1, 3)
                idx_wait(g + 1, rn)
                pltpu.async_copy(xh_hbm.at[sidx.at[rn]], rows2.at[pn],
                                 gsem2.at[pn])

                @pl.when(g + 2 < NCH)
                def _():
                    idx_load(g + 2, lax.rem(g + 2, 3))

            pltpu.make_async_copy(xh_hbm.at[sidx.at[r]], rows2.at[p],
                                  gsem2.at[p]).wait()
            pltpu.async_copy(rows2.at[p], s_sh.at[didx.at[r]], ssem2.at[p],
                             add=True)
            return carry

        lax.fori_loop(0, NCH, chunk, 0)
        pltpu.make_async_copy(rows2.at[(NCH - 1) % 2],
                              s_sh.at[didx.at[(NCH - 1) % 3]],
                              ssem2.at[(NCH - 1) % 2]).wait()
        plsc.subcore_barrier()

        pltpu.sync_copy(s_sh.at[pl.ds(rbase, RPW)],
                        out_hbm.at[pl.ds(c * NPH + rbase, RPW)])


_spmv = functools.partial(
    pl.kernel,
    out_type=(jax.ShapeDtypeStruct((NP, KH), jnp.float32),
              jax.ShapeDtypeStruct((NP, KH), jnp.float32)),
    mesh=_MESH,
    compiler_params=_SC_PARAMS,
    scratch_types=[
        pltpu.VMEM((3, CH), jnp.int32),
        pltpu.VMEM((3, CH), jnp.int32),
        pltpu.VMEM((2, CH, KH), jnp.float32),
        pltpu.VMEM_SHARED((NPH, KH), jnp.float32),
        pltpu.SemaphoreType.DMA((2,)),
        pltpu.SemaphoreType.DMA((2,)),
        pltpu.SemaphoreType.DMA((3,)),
    ],
)(_spmv_body)


def _gather_body(t0, t1, t2, t3, t4, t5, t6, t7, uid_hbm, iid_hbm,
                 gu_hbm, gi_hbm, idx2, rows2, sem2):
    c = lax.axis_index("c")
    t = lax.axis_index("s")
    wid = t * NC + c
    base = wid * BPW

    pltpu.sync_copy(uid_hbm.at[pl.ds(base, BPW)], idx2.at[0])
    pltpu.sync_copy(iid_hbm.at[pl.ds(base, BPW)], idx2.at[1])

    tabs = (t0, t1, t2, t3, t4, t5, t6, t7)
    outs = (gu_hbm, gi_hbm)
    ops = [(s, l) for s in (0, 1) for l in range(len(tabs))]
    prev = None
    for k, (s, l) in enumerate(ops):
        p = k % 2
        pltpu.async_copy(tabs[l].at[idx2.at[s]], rows2.at[p], sem2.at[p])
        if prev is not None:
            qs, ql, qp = prev
            pltpu.make_async_copy(tabs[ql].at[idx2.at[qs]], rows2.at[qp],
                                  sem2.at[qp]).wait()
            pltpu.sync_copy(rows2.at[qp], outs[qs].at[ql, pl.ds(base, BPW)])
        prev = (s, l, p)
    qs, ql, qp = prev
    pltpu.make_async_copy(tabs[ql].at[idx2.at[qs]], rows2.at[qp],
                          sem2.at[qp]).wait()
    pltpu.sync_copy(rows2.at[qp], outs[qs].at[ql, pl.ds(base, BPW)])


_gather = functools.partial(
    pl.kernel,
    out_type=(jax.ShapeDtypeStruct((8, BATCH, KH), jnp.float32),
              jax.ShapeDtypeStruct((8, BATCH, KH), jnp.float32)),
    mesh=_MESH,
    compiler_params=_SC_PARAMS,
    scratch_types=[
        pltpu.VMEM((2, BPW), jnp.int32),
        pltpu.VMEM((2, BPW, KH), jnp.float32),
        pltpu.SemaphoreType.DMA((2,)),
    ],
)(_gather_body)


BLKR = 1568


def _dense_body(xa_ref, xb_ref, sa_ref, sb_ref,
                w1aa, w1ba, w1ab, w1bb, w2aa, w2ba, w2ab, w2bb, onesbd,
                ha_ref, hb_ref):
    xa = xa_ref[...]
    xb = xb_ref[...]
    sa = sa_ref[...]
    sb = sb_ref[...]
    ta = xa + sa
    tb = xb + sb
    ma = xa * sa
    mb = xb * sb

    def mm(a, b):
        return jnp.dot(a, b, preferred_element_type=jnp.float32)

    ha = mm(ta, w1aa[...]) + mm(tb, w1ba[...]) + mm(ma, w2aa[...]) \
        + mm(mb, w2ba[...])
    hb = mm(ta, w1ab[...]) + mm(tb, w1bb[...]) + mm(ma, w2ab[...]) \
        + mm(mb, w2bb[...])
    ha = jnp.where(ha > 0, ha, 0.2 * ha)
    hb = jnp.where(hb > 0, hb, 0.2 * hb)
    n2 = mm(ha * ha + hb * hb, onesbd[...])
    inv = 1.0 / jnp.maximum(jnp.sqrt(n2), 1e-12)
    ha_ref[...] = ha * inv
    hb_ref[...] = hb * inv


_prow_spec = pl.BlockSpec((BLKR, 128), lambda i: (i, 0))
_w_spec = pl.BlockSpec((128, 128), lambda i: (0, 0))
_dense = pl.pallas_call(
    _dense_body,
    grid=(PR // BLKR,),
    in_specs=[_prow_spec] * 4 + [_w_spec] * 9,
    out_specs=(_prow_spec, _prow_spec),
    out_shape=(jax.ShapeDtypeStruct((PR, 128), jnp.float32),
               jax.ShapeDtypeStruct((PR, 128), jnp.float32)),
)

BBLK = 2048


def _finale_body(gu_ref, gi_ref, o_ref, gau_ref, gai_ref):
    gu = jnp.concatenate([gu_ref[l] for l in range(8)], axis=1)
    gi = jnp.concatenate([gi_ref[l] for l in range(8)], axis=1)
    o_ref[...] = jnp.sum(gu * gi, axis=1)
    gau_ref[...] = gu
    gai_ref[...] = gi


_finale = pl.pallas_call(
    _finale_body,
    grid=(BATCH // BBLK,),
    in_specs=[
        pl.BlockSpec((8, BBLK, KH), lambda i: (0, i, 0)),
        pl.BlockSpec((8, BBLK, KH), lambda i: (0, i, 0)),
    ],
    out_specs=(pl.BlockSpec((BBLK,), lambda i: (i,)),
               pl.BlockSpec((BBLK, 8 * KH), lambda i: (i, 0)),
               pl.BlockSpec((BBLK, 8 * KH), lambda i: (i, 0))),
    out_shape=(jax.ShapeDtypeStruct((BATCH,), jnp.float32),
               jax.ShapeDtypeStruct((BATCH, 8 * KH), jnp.float32),
               jax.ShapeDtypeStruct((BATCH, 8 * KH), jnp.float32)),
)


def kernel(Gu, Gi, edge_index, users, items,
           W1_0, W2_0, W1_1, W2_1, W1_2, W2_2):
    f32 = jnp.float32
    e0 = edge_index[0].astype(jnp.int32)
    e1 = edge_index[1].astype(jnp.int32)
    src2 = jnp.concatenate([e1 + NPH, e0])
    dst2 = jnp.concatenate([e0, e1])
    zeros_pad = jnp.zeros((RPW, KH), f32)

    pad = jnp.zeros((NPH - NU, K), f32)
    x0 = jnp.concatenate([Gu, pad, Gi, pad], axis=0)
    xa = x0[:, :KH].reshape(PR, 128)
    xb = x0[:, KH:].reshape(PR, 128)

    eye4 = jnp.eye(4, dtype=f32)

    def bd(m):
        return jnp.kron(eye4, m)

    onesbd = bd(jnp.ones((KH, KH), f32))

    tabs = [xa, xb]
    for (W1, W2) in ((W1_0, W2_0), (W1_1, W2_1), (W1_2, W2_2)):
        sa, sb = _spmv(src2, dst2, xa.reshape(NP, KH), xb.reshape(NP, KH),
                       zeros_pad)
        xa, xb = _dense(xa, xb, sa.reshape(PR, 128), sb.reshape(PR, 128),
                        bd(W1[:KH, :KH]), bd(W1[KH:, :KH]),
                        bd(W1[:KH, KH:]), bd(W1[KH:, KH:]),
                        bd(W2[:KH, :KH]), bd(W2[KH:, :KH]),
                        bd(W2[:KH, KH:]), bd(W2[KH:, KH:]), onesbd)
        tabs += [xa, xb]

    uid = users.astype(jnp.int32)
    iid = items.astype(jnp.int32) + NPH
    gu8, gi8 = _gather(*[tb.reshape(NP, KH) for tb in tabs], uid, iid)
    xui, gamma_u, gamma_i = _finale(gu8, gi8)
    return (xui, gamma_u, gamma_i)


# speedup vs baseline: 26.6168x; 1.0878x over previous
import functools

import jax
import jax.numpy as jnp
from jax import lax
from jax.experimental import pallas as pl
from jax.experimental.pallas import tpu as pltpu
from jax.experimental.pallas import tpu_sc as plsc

NU = 25000
NI = 25000
K = 64
KH = 32
NE = 400000
BATCH = 16384

NC = 2
NS = 16
EPW = NE // NS
CH = 1000
NCH = EPW // CH
RPW = 1568
NPH = RPW * NS
NP = 2 * NPH
PR = NP * KH // 128

BPW = BATCH // (NC * NS)

_MESH = plsc.VectorSubcoreMesh(core_axis_name="c", subcore_axis_name="s")
_SC_PARAMS = pltpu.CompilerParams(use_tc_tiling_on_sc=False)


def _spmv_body(src_hbm, dst_hbm, xa_hbm, xb_hbm, zero_hbm, sa_hbm, sb_hbm,
               sidx, didx, rows2, s_sh, gsem2, ssem2, isem3):
    c = lax.axis_index("c")
    t = lax.axis_index("s")
    rbase = t * RPW
    ebase = c * NE + t * EPW

    def idx_load(g, slot):
        pltpu.async_copy(src_hbm.at[pl.ds(ebase + g * CH, CH)],
                         sidx.at[slot], isem3.at[slot])
        pltpu.async_copy(dst_hbm.at[pl.ds(ebase + g * CH, CH)],
                         didx.at[slot], isem3.at[slot])

    def idx_wait(g, slot):
        pltpu.make_async_copy(src_hbm.at[pl.ds(ebase + g * CH, CH)],
                              sidx.at[slot], isem3.at[slot]).wait()
        pltpu.make_async_copy(dst_hbm.at[pl.ds(ebase + g * CH, CH)],
                              didx.at[slot], isem3.at[slot]).wait()

    for xh_hbm, out_hbm in ((xa_hbm, sa_hbm), (xb_hbm, sb_hbm)):
        pltpu.sync_copy(zero_hbm, s_sh.at[pl.ds(rbase, RPW)])
        plsc.subcore_barrier()

        idx_load(0, 0)
        idx_load(1, 1)
        idx_wait(0, 0)
        pltpu.async_copy(xh_hbm.at[sidx.at[0]], rows2.at[0], gsem2.at[0])

        def chunk(g, carry):
            p = lax.rem(g, 2)
            pn = 1 - p
            r = lax.rem(g, 3)

            @pl.when(g >= 1)
            def _():
                pltpu.make_async_copy(rows2.at[pn],
                                      s_sh.at[didx.at[lax.rem(g + 2, 3)]],
                                      ssem2.at[pn]).wait()

            @pl.when(g + 1 < NCH)
            def _():
                rn = lax.rem(g + 1, 3)
                idx_wait(g + 1, rn)
                pltpu.async_copy(xh_hbm.at[sidx.at[rn]], rows2.at[pn],
                                 gsem2.at[pn])

                @pl.when(g + 2 < NCH)
                def _():
                    idx_load(g + 2, lax.rem(g + 2, 3))

            pltpu.make_async_copy(xh_hbm.at[sidx.at[r]], rows2.at[p],
                                  gsem2.at[p]).wait()
            pltpu.async_copy(rows2.at[p], s_sh.at[didx.at[r]], ssem2.at[p],
                             add=True)
            return carry

        lax.fori_loop(0, NCH, chunk, 0)
        pltpu.make_async_copy(rows2.at[(NCH - 1) % 2],
                              s_sh.at[didx.at[(NCH - 1) % 3]],
                              ssem2.at[(NCH - 1) % 2]).wait()
        plsc.subcore_barrier()

        pltpu.sync_copy(s_sh.at[pl.ds(rbase, RPW)],
                        out_hbm.at[pl.ds(c * NPH + rbase, RPW)])


_spmv = functools.partial(
    pl.kernel,
    out_type=(jax.ShapeDtypeStruct((NP, KH), jnp.float32),
              jax.ShapeDtypeStruct((NP, KH), jnp.float32)),
    mesh=_MESH,
    compiler_params=_SC_PARAMS,
    scratch_types=[
        pltpu.VMEM((3, CH), jnp.int32),
        pltpu.VMEM((3, CH), jnp.int32),
        pltpu.VMEM((2, CH, KH), jnp.float32),
        pltpu.VMEM_SHARED((NPH, KH), jnp.float32),
        pltpu.SemaphoreType.DMA((2,)),
        pltpu.SemaphoreType.DMA((2,)),
        pltpu.SemaphoreType.DMA((3,)),
    ],
)(_spmv_body)


def _gather_body(t0, t1, t2, t3, t4, t5, t6, t7, uid_hbm, iid_hbm,
                 gu_hbm, gi_hbm, idx2, rows2, sem2):
    c = lax.axis_index("c")
    t = lax.axis_index("s")
    wid = t * NC + c
    base = wid * BPW

    pltpu.sync_copy(uid_hbm.at[pl.ds(base, BPW)], idx2.at[0])
    pltpu.sync_copy(iid_hbm.at[pl.ds(base, BPW)], idx2.at[1])

    tabs = (t0, t1, t2, t3, t4, t5, t6, t7)
    outs = (gu_hbm, gi_hbm)
    ops = [(s, l) for s in (0, 1) for l in range(len(tabs))]

    def write(qs, ql, qp):
        pltpu.make_async_copy(tabs[ql].at[idx2.at[qs]], rows2.at[qp],
                              sem2.at[qp]).wait()
        pltpu.sync_copy(rows2.at[qp],
                        outs[qs].at[pl.ds(base, BPW), pl.ds(ql * KH, KH)])

    prev = None
    for k, (s, l) in enumerate(ops):
        p = k % 2
        pltpu.async_copy(tabs[l].at[idx2.at[s]], rows2.at[p], sem2.at[p])
        if prev is not None:
            write(*prev)
        prev = (s, l, p)
    write(*prev)


_gather = functools.partial(
    pl.kernel,
    out_type=(jax.ShapeDtypeStruct((BATCH, 8 * KH), jnp.float32),
              jax.ShapeDtypeStruct((BATCH, 8 * KH), jnp.float32)),
    mesh=_MESH,
    compiler_params=_SC_PARAMS,
    scratch_types=[
        pltpu.VMEM((2, BPW), jnp.int32),
        pltpu.VMEM((2, BPW, KH), jnp.float32),
        pltpu.SemaphoreType.DMA((2,)),
    ],
)(_gather_body)


BLKR = 1568


def _dense_body(xa_ref, xb_ref, sa_ref, sb_ref,
                w1aa, w1ba, w1ab, w1bb, w2aa, w2ba, w2ab, w2bb, onesbd,
                ha_ref, hb_ref):
    xa = xa_ref[...]
    xb = xb_ref[...]
    sa = sa_ref[...]
    sb = sb_ref[...]
    ta = xa + sa
    tb = xb + sb
    ma = xa * sa
    mb = xb * sb

    def mm(a, b):
        return jnp.dot(a, b, preferred_element_type=jnp.float32)

    ha = mm(ta, w1aa[...]) + mm(tb, w1ba[...]) + mm(ma, w2aa[...]) \
        + mm(mb, w2ba[...])
    hb = mm(ta, w1ab[...]) + mm(tb, w1bb[...]) + mm(ma, w2ab[...]) \
        + mm(mb, w2bb[...])
    ha = jnp.where(ha > 0, ha, 0.2 * ha)
    hb = jnp.where(hb > 0, hb, 0.2 * hb)
    n2 = mm(ha * ha + hb * hb, onesbd[...])
    inv = 1.0 / jnp.maximum(jnp.sqrt(n2), 1e-12)
    ha_ref[...] = ha * inv
    hb_ref[...] = hb * inv


_prow_spec = pl.BlockSpec((BLKR, 128), lambda i: (i, 0))
_w_spec = pl.BlockSpec((128, 128), lambda i: (0, 0))
_dense = pl.pallas_call(
    _dense_body,
    grid=(PR // BLKR,),
    in_specs=[_prow_spec] * 4 + [_w_spec] * 9,
    out_specs=(_prow_spec, _prow_spec),
    out_shape=(jax.ShapeDtypeStruct((PR, 128), jnp.float32),
               jax.ShapeDtypeStruct((PR, 128), jnp.float32)),
)

GROWS = BATCH * 8 * KH // 128
BBLK = 4096


def _xui_body(gu_ref, gi_ref, o_ref):
    o_ref[...] = jnp.sum(gu_ref[...] * gi_ref[...], axis=1)


_xui = pl.pallas_call(
    _xui_body,
    grid=(GROWS // BBLK,),
    in_specs=[
        pl.BlockSpec((BBLK, 128), lambda i: (i, 0)),
        pl.BlockSpec((BBLK, 128), lambda i: (i, 0)),
    ],
    out_specs=pl.BlockSpec((BBLK,), lambda i: (i,)),
    out_shape=jax.ShapeDtypeStruct((GROWS,), jnp.float32),
)


def kernel(Gu, Gi, edge_index, users, items,
           W1_0, W2_0, W1_1, W2_1, W1_2, W2_2):
    f32 = jnp.float32
    e0 = edge_index[0].astype(jnp.int32)
    e1 = edge_index[1].astype(jnp.int32)
    src2 = jnp.concatenate([e1 + NPH, e0])
    dst2 = jnp.concatenate([e0, e1])
    zeros_pad = jnp.zeros((RPW, KH), f32)

    padh = jnp.zeros((NPH - NU, KH), f32)
    xa = jnp.concatenate([Gu[:, :KH], padh, Gi[:, :KH], padh],
                         axis=0).reshape(PR, 128)
    xb = jnp.concatenate([Gu[:, KH:], padh, Gi[:, KH:], padh],
                         axis=0).reshape(PR, 128)

    eye4 = jnp.eye(4, dtype=f32)

    def bd(m):
        return jnp.kron(eye4, m)

    onesbd = bd(jnp.ones((KH, KH), f32))

    tabs = [xa, xb]
    for (W1, W2) in ((W1_0, W2_0), (W1_1, W2_1), (W1_2, W2_2)):
        sa, sb = _spmv(src2, dst2, xa.reshape(NP, KH), xb.reshape(NP, KH),
                       zeros_pad)
        xa, xb = _dense(xa, xb, sa.reshape(PR, 128), sb.reshape(PR, 128),
                        bd(W1[:KH, :KH]), bd(W1[KH:, :KH]),
                        bd(W1[:KH, KH:]), bd(W1[KH:, KH:]),
                        bd(W2[:KH, :KH]), bd(W2[KH:, :KH]),
                        bd(W2[:KH, KH:]), bd(W2[KH:, KH:]), onesbd)
        tabs += [xa, xb]

    uid = users.astype(jnp.int32)
    iid = items.astype(jnp.int32) + NPH
    gamma_u, gamma_i = _gather(*[tb.reshape(NP, KH) for tb in tabs],
                               uid, iid)
    part = _xui(gamma_u.reshape(GROWS, 128), gamma_i.reshape(GROWS, 128))
    xui = part.reshape(BATCH, 2).sum(axis=1)
    return (xui, gamma_u, gamma_i)


# speedup vs baseline: 26.9416x; 1.0122x over previous
import functools

import jax
import jax.numpy as jnp
from jax import lax
from jax.experimental import pallas as pl
from jax.experimental.pallas import tpu as pltpu
from jax.experimental.pallas import tpu_sc as plsc

NU = 25000
NI = 25000
K = 64
KH = 32
NE = 400000
BATCH = 16384

NC = 2
NS = 16
EPW = NE // NS
CH = 1000
NCH = EPW // CH
RPW = 1568
NPH = RPW * NS
NP = 2 * NPH
PR = NP * KH // 128

BPW = BATCH // (NC * NS)

_MESH = plsc.VectorSubcoreMesh(core_axis_name="c", subcore_axis_name="s")
_SC_PARAMS = pltpu.CompilerParams(use_tc_tiling_on_sc=False)


def _spmv_body(src_hbm, dst_hbm, xa_hbm, xb_hbm, zero_hbm, sa_hbm, sb_hbm,
               sidx, didx, rows2, s_sh, gsem2, ssem2, isem3):
    c = lax.axis_index("c")
    t = lax.axis_index("s")
    rbase = t * RPW
    ebase = c * NE + t * EPW

    def idx_load(g, slot):
        pltpu.async_copy(src_hbm.at[pl.ds(ebase + g * CH, CH)],
                         sidx.at[slot], isem3.at[slot])
        pltpu.async_copy(dst_hbm.at[pl.ds(ebase + g * CH, CH)],
                         didx.at[slot], isem3.at[slot])

    def idx_wait(g, slot):
        pltpu.make_async_copy(src_hbm.at[pl.ds(ebase + g * CH, CH)],
                              sidx.at[slot], isem3.at[slot]).wait()
        pltpu.make_async_copy(dst_hbm.at[pl.ds(ebase + g * CH, CH)],
                              didx.at[slot], isem3.at[slot]).wait()

    for xh_hbm, out_hbm in ((xa_hbm, sa_hbm), (xb_hbm, sb_hbm)):
        pltpu.sync_copy(zero_hbm, s_sh.at[pl.ds(rbase, RPW)])
        plsc.subcore_barrier()

        idx_load(0, 0)
        idx_load(1, 1)
        idx_wait(0, 0)
        pltpu.async_copy(xh_hbm.at[sidx.at[0]], rows2.at[0], gsem2.at[0])

        def chunk(g, carry):
            p = lax.rem(g, 2)
            pn = 1 - p
            r = lax.rem(g, 3)

            @pl.when(g >= 1)
            def _():
                pltpu.make_async_copy(rows2.at[pn],
                                      s_sh.at[didx.at[lax.rem(g + 2, 3)]],
                                      ssem2.at[pn]).wait()

            @pl.when(g + 1 < NCH)
            def _():
                rn = lax.rem(g + 1, 3)
                idx_wait(g + 1, rn)
                pltpu.async_copy(xh_hbm.at[sidx.at[rn]], rows2.at[pn],
                                 gsem2.at[pn])

                @pl.when(g + 2 < NCH)
                def _():
                    idx_load(g + 2, lax.rem(g + 2, 3))

            pltpu.make_async_copy(xh_hbm.at[sidx.at[r]], rows2.at[p],
                                  gsem2.at[p]).wait()
            pltpu.async_copy(rows2.at[p], s_sh.at[didx.at[r]], ssem2.at[p],
                             add=True)
            return carry

        lax.fori_loop(0, NCH, chunk, 0)
        pltpu.make_async_copy(rows2.at[(NCH - 1) % 2],
                              s_sh.at[didx.at[(NCH - 1) % 3]],
                              ssem2.at[(NCH - 1) % 2]).wait()
        plsc.subcore_barrier()

        pltpu.sync_copy(s_sh.at[pl.ds(rbase, RPW)],
                        out_hbm.at[pl.ds(c * NPH + rbase, RPW)])


_spmv = functools.partial(
    pl.kernel,
    out_type=(jax.ShapeDtypeStruct((NP, KH), jnp.float32),
              jax.ShapeDtypeStruct((NP, KH), jnp.float32)),
    mesh=_MESH,
    compiler_params=_SC_PARAMS,
    scratch_types=[
        pltpu.VMEM((3, CH), jnp.int32),
        pltpu.VMEM((3, CH), jnp.int32),
        pltpu.VMEM((2, CH, KH), jnp.float32),
        pltpu.VMEM_SHARED((NPH, KH), jnp.float32),
        pltpu.SemaphoreType.DMA((2,)),
        pltpu.SemaphoreType.DMA((2,)),
        pltpu.SemaphoreType.DMA((3,)),
    ],
)(_spmv_body)


def _gather_body(t0, t1, t2, t3, t4, t5, t6, t7, uid_hbm, iid_hbm,
                 gu_hbm, gi_hbm, idx2, rows2, sem2):
    c = lax.axis_index("c")
    t = lax.axis_index("s")
    wid = t * NC + c
    base = wid * BPW

    pltpu.sync_copy(uid_hbm.at[pl.ds(base, BPW)], idx2.at[0])
    pltpu.sync_copy(iid_hbm.at[pl.ds(base, BPW)], idx2.at[1])

    tabs = (t0, t1, t2, t3, t4, t5, t6, t7)
    outs = (gu_hbm, gi_hbm)
    ops = [(s, l) for s in (0, 1) for l in range(len(tabs))]

    def write(qs, ql, qp):
        pltpu.make_async_copy(tabs[ql].at[idx2.at[qs]], rows2.at[qp],
                              sem2.at[qp]).wait()
        pltpu.sync_copy(rows2.at[qp],
                        outs[qs].at[pl.ds(base, BPW), pl.ds(ql * KH, KH)])

    prev = None
    for k, (s, l) in enumerate(ops):
        p = k % 2
        pltpu.async_copy(tabs[l].at[idx2.at[s]], rows2.at[p], sem2.at[p])
        if prev is not None:
            write(*prev)
        prev = (s, l, p)
    write(*prev)


_gather = functools.partial(
    pl.kernel,
    out_type=(jax.ShapeDtypeStruct((BATCH, 8 * KH), jnp.float32),
              jax.ShapeDtypeStruct((BATCH, 8 * KH), jnp.float32)),
    mesh=_MESH,
    compiler_params=_SC_PARAMS,
    scratch_types=[
        pltpu.VMEM((2, BPW), jnp.int32),
        pltpu.VMEM((2, BPW, KH), jnp.float32),
        pltpu.SemaphoreType.DMA((2,)),
    ],
)(_gather_body)


BLKR = 1568


def _dense_body(xa_ref, xb_ref, sa_ref, sb_ref,
                w1aa, w1ba, w1ab, w1bb, w2aa, w2ba, w2ab, w2bb, onesbd,
                ha_ref, hb_ref):
    xa = xa_ref[...]
    xb = xb_ref[...]
    sa = sa_ref[...]
    sb = sb_ref[...]
    ta = xa + sa
    tb = xb + sb
    ma = xa * sa
    mb = xb * sb

    def mm(a, b):
        return jnp.dot(a, b, preferred_element_type=jnp.float32)

    ha = mm(ta, w1aa[...]) + mm(tb, w1ba[...]) + mm(ma, w2aa[...]) \
        + mm(mb, w2ba[...])
    hb = mm(ta, w1ab[...]) + mm(tb, w1bb[...]) + mm(ma, w2ab[...]) \
        + mm(mb, w2bb[...])
    ha = jnp.where(ha > 0, ha, 0.2 * ha)
    hb = jnp.where(hb > 0, hb, 0.2 * hb)
    n2 = mm(ha * ha + hb * hb, onesbd[...])
    inv = 1.0 / jnp.maximum(jnp.sqrt(n2), 1e-12)
    ha_ref[...] = ha * inv
    hb_ref[...] = hb * inv


_prow_spec = pl.BlockSpec((BLKR, 128), lambda i: (i, 0))
_w_spec = pl.BlockSpec((128, 128), lambda i: (0, 0))
_dense = pl.pallas_call(
    _dense_body,
    grid=(PR // BLKR,),
    in_specs=[_prow_spec] * 4 + [_w_spec] * 9,
    out_specs=(_prow_spec, _prow_spec),
    out_shape=(jax.ShapeDtypeStruct((PR, 128), jnp.float32),
               jax.ShapeDtypeStruct((PR, 128), jnp.float32)),
)

BBLK = 2048


def _xui_body(gu_ref, gi_ref, o_ref):
    o_ref[...] = jnp.sum(gu_ref[...] * gi_ref[...], axis=1)


_xui = pl.pallas_call(
    _xui_body,
    grid=(BATCH // BBLK,),
    in_specs=[
        pl.BlockSpec((BBLK, 8 * KH), lambda i: (i, 0)),
        pl.BlockSpec((BBLK, 8 * KH), lambda i: (i, 0)),
    ],
    out_specs=pl.BlockSpec((BBLK,), lambda i: (i,)),
    out_shape=jax.ShapeDtypeStruct((BATCH,), jnp.float32),
)


def kernel(Gu, Gi, edge_index, users, items,
           W1_0, W2_0, W1_1, W2_1, W1_2, W2_2):
    f32 = jnp.float32
    e0 = edge_index[0].astype(jnp.int32)
    e1 = edge_index[1].astype(jnp.int32)
    src2 = jnp.concatenate([e1 + NPH, e0])
    dst2 = jnp.concatenate([e0, e1])
    zeros_pad = jnp.zeros((RPW, KH), f32)

    npp = NU * KH // 128
    zp = jnp.zeros(((NPH - NU) * KH // 128, 128), f32)
    xa = jnp.concatenate([Gu[:, :KH].reshape(npp, 128), zp,
                          Gi[:, :KH].reshape(npp, 128), zp], axis=0)
    xb = jnp.concatenate([Gu[:, KH:].reshape(npp, 128), zp,
                          Gi[:, KH:].reshape(npp, 128), zp], axis=0)

    eye4 = jnp.eye(4, dtype=f32)

    def bd(m):
        return jnp.kron(eye4, m)

    onesbd = bd(jnp.ones((KH, KH), f32))

    tabs = [xa, xb]
    for (W1, W2) in ((W1_0, W2_0), (W1_1, W2_1), (W1_2, W2_2)):
        sa, sb = _spmv(src2, dst2, xa.reshape(NP, KH), xb.reshape(NP, KH),
                       zeros_pad)
        xa, xb = _dense(xa, xb, sa.reshape(PR, 128), sb.reshape(PR, 128),
                        bd(W1[:KH, :KH]), bd(W1[KH:, :KH]),
                        bd(W1[:KH, KH:]), bd(W1[KH:, KH:]),
                        bd(W2[:KH, :KH]), bd(W2[KH:, :KH]),
                        bd(W2[:KH, KH:]), bd(W2[KH:, KH:]), onesbd)
        tabs += [xa, xb]

    uid = users.astype(jnp.int32)
    iid = items.astype(jnp.int32) + NPH
    gamma_u, gamma_i = _gather(*[tb.reshape(NP, KH) for tb in tabs],
                               uid, iid)
    xui = _xui(gamma_u, gamma_i)
    return (xui, gamma_u, gamma_i)


# speedup vs baseline: 27.0753x; 1.0050x over previous
import functools

import jax
import jax.numpy as jnp
from jax import lax
from jax.experimental import pallas as pl
from jax.experimental.pallas import tpu as pltpu
from jax.experimental.pallas import tpu_sc as plsc

NU = 25000
NI = 25000
K = 64
KH = 32
NE = 400000
BATCH = 16384

NC = 2
NS = 16
EPW = NE // NS
CH = 1000
NCH = EPW // CH
RPW = 1568
NPH = RPW * NS
NP = 2 * NPH
PR = NP * KH // 128

BPW = BATCH // (NC * NS)

_MESH = plsc.VectorSubcoreMesh(core_axis_name="c", subcore_axis_name="s")
_SC_PARAMS = pltpu.CompilerParams(use_tc_tiling_on_sc=False)


def _spmv_body(src_hbm, dst_hbm, xa_hbm, xb_hbm, zero_hbm, sa_hbm, sb_hbm,
               sidx, didx, rows2, s_sh, gsem2, ssem2, isem3):
    c = lax.axis_index("c")
    t = lax.axis_index("s")
    rbase = t * RPW
    ebase = c * NE + t * EPW

    def idx_load(g, slot):
        pltpu.async_copy(src_hbm.at[pl.ds(ebase + g * CH, CH)],
                         sidx.at[slot], isem3.at[slot])
        pltpu.async_copy(dst_hbm.at[pl.ds(ebase + g * CH, CH)],
                         didx.at[slot], isem3.at[slot])

    def idx_wait(g, slot):
        pltpu.make_async_copy(src_hbm.at[pl.ds(ebase + g * CH, CH)],
                              sidx.at[slot], isem3.at[slot]).wait()
        pltpu.make_async_copy(dst_hbm.at[pl.ds(ebase + g * CH, CH)],
                              didx.at[slot], isem3.at[slot]).wait()

    for xh_hbm, out_hbm in ((xa_hbm, sa_hbm), (xb_hbm, sb_hbm)):
        pltpu.sync_copy(zero_hbm, s_sh.at[pl.ds(rbase, RPW)])
        plsc.subcore_barrier()

        idx_load(0, 0)
        idx_load(1, 1)
        idx_wait(0, 0)
        pltpu.async_copy(xh_hbm.at[sidx.at[0]], rows2.at[0], gsem2.at[0])

        def chunk(g, carry):
            p = lax.rem(g, 2)
            pn = 1 - p
            r = lax.rem(g, 3)

            @pl.when(g >= 1)
            def _():
                pltpu.make_async_copy(rows2.at[pn],
                                      s_sh.at[didx.at[lax.rem(g + 2, 3)]],
                                      ssem2.at[pn]).wait()

            @pl.when(g + 1 < NCH)
            def _():
                rn = lax.rem(g + 1, 3)
                idx_wait(g + 1, rn)
                pltpu.async_copy(xh_hbm.at[sidx.at[rn]], rows2.at[pn],
                                 gsem2.at[pn])

                @pl.when(g + 2 < NCH)
                def _():
                    idx_load(g + 2, lax.rem(g + 2, 3))

            pltpu.make_async_copy(xh_hbm.at[sidx.at[r]], rows2.at[p],
                                  gsem2.at[p]).wait()
            pltpu.async_copy(rows2.at[p], s_sh.at[didx.at[r]], ssem2.at[p],
                             add=True)
            return carry

        lax.fori_loop(0, NCH, chunk, 0)
        pltpu.make_async_copy(rows2.at[(NCH - 1) % 2],
                              s_sh.at[didx.at[(NCH - 1) % 3]],
                              ssem2.at[(NCH - 1) % 2]).wait()
        plsc.subcore_barrier()

        pltpu.sync_copy(s_sh.at[pl.ds(rbase, RPW)],
                        out_hbm.at[pl.ds(c * NPH + rbase, RPW)])


_spmv = functools.partial(
    pl.kernel,
    out_type=(jax.ShapeDtypeStruct((NP, KH), jnp.float32),
              jax.ShapeDtypeStruct((NP, KH), jnp.float32)),
    mesh=_MESH,
    compiler_params=_SC_PARAMS,
    scratch_types=[
        pltpu.VMEM((3, CH), jnp.int32),
        pltpu.VMEM((3, CH), jnp.int32),
        pltpu.VMEM((2, CH, KH), jnp.float32),
        pltpu.VMEM_SHARED((NPH, KH), jnp.float32),
        pltpu.SemaphoreType.DMA((2,)),
        pltpu.SemaphoreType.DMA((2,)),
        pltpu.SemaphoreType.DMA((3,)),
    ],
)(_spmv_body)


def _gather_body(t0, t1, t2, t3, t4, t5, t6, t7, uid_hbm, iid_hbm,
                 gu_hbm, gi_hbm, idx2, rows2, sem2):
    c = lax.axis_index("c")
    t = lax.axis_index("s")
    wid = t * NC + c
    base = wid * BPW

    pltpu.sync_copy(uid_hbm.at[pl.ds(base, BPW)], idx2.at[0])
    pltpu.sync_copy(iid_hbm.at[pl.ds(base, BPW)], idx2.at[1])

    tabs = (t0, t1, t2, t3, t4, t5, t6, t7)
    outs = (gu_hbm, gi_hbm)
    ops = [(s, l) for s in (0, 1) for l in range(len(tabs))]

    def write(qs, ql, qp):
        pltpu.make_async_copy(tabs[ql].at[idx2.at[qs]], rows2.at[qp],
                              sem2.at[qp]).wait()
        pltpu.sync_copy(rows2.at[qp],
                        outs[qs].at[pl.ds(base, BPW), pl.ds(ql * KH, KH)])

    prev = None
    for k, (s, l) in enumerate(ops):
        p = k % 2
        pltpu.async_copy(tabs[l].at[idx2.at[s]], rows2.at[p], sem2.at[p])
        if prev is not None:
            write(*prev)
        prev = (s, l, p)
    write(*prev)


_gather = functools.partial(
    pl.kernel,
    out_type=(jax.ShapeDtypeStruct((BATCH, 8 * KH), jnp.float32),
              jax.ShapeDtypeStruct((BATCH, 8 * KH), jnp.float32)),
    mesh=_MESH,
    compiler_params=_SC_PARAMS,
    scratch_types=[
        pltpu.VMEM((2, BPW), jnp.int32),
        pltpu.VMEM((2, BPW, KH), jnp.float32),
        pltpu.SemaphoreType.DMA((2,)),
    ],
)(_gather_body)


BLKR = 3136


def _dense_body(xa_ref, xb_ref, sa_ref, sb_ref,
                w1aa, w1ba, w1ab, w1bb, w2aa, w2ba, w2ab, w2bb, onesbd,
                ha_ref, hb_ref):
    xa = xa_ref[...]
    xb = xb_ref[...]
    sa = sa_ref[...]
    sb = sb_ref[...]
    ta = xa + sa
    tb = xb + sb
    ma = xa * sa
    mb = xb * sb

    def mm(a, b):
        return jnp.dot(a, b, preferred_element_type=jnp.float32)

    ha = mm(ta, w1aa[...]) + mm(tb, w1ba[...]) + mm(ma, w2aa[...]) \
        + mm(mb, w2ba[...])
    hb = mm(ta, w1ab[...]) + mm(tb, w1bb[...]) + mm(ma, w2ab[...]) \
        + mm(mb, w2bb[...])
    ha = jnp.where(ha > 0, ha, 0.2 * ha)
    hb = jnp.where(hb > 0, hb, 0.2 * hb)
    n2 = mm(ha * ha + hb * hb, onesbd[...])
    inv = 1.0 / jnp.maximum(jnp.sqrt(n2), 1e-12)
    ha_ref[...] = ha * inv
    hb_ref[...] = hb * inv


_prow_spec = pl.BlockSpec((BLKR, 128), lambda i: (i, 0))
_w_spec = pl.BlockSpec((128, 128), lambda i: (0, 0))
_dense = pl.pallas_call(
    _dense_body,
    grid=(PR // BLKR,),
    in_specs=[_prow_spec] * 4 + [_w_spec] * 9,
    out_specs=(_prow_spec, _prow_spec),
    out_shape=(jax.ShapeDtypeStruct((PR, 128), jnp.float32),
               jax.ShapeDtypeStruct((PR, 128), jnp.float32)),
)

BBLK = 2048


def _xui_body(gu_ref, gi_ref, o_ref):
    o_ref[...] = jnp.sum(gu_ref[...] * gi_ref[...], axis=1)


_xui = pl.pallas_call(
    _xui_body,
    grid=(BATCH // BBLK,),
    in_specs=[
        pl.BlockSpec((BBLK, 8 * KH), lambda i: (i, 0)),
        pl.BlockSpec((BBLK, 8 * KH), lambda i: (i, 0)),
    ],
    out_specs=pl.BlockSpec((BBLK,), lambda i: (i,)),
    out_shape=jax.ShapeDtypeStruct((BATCH,), jnp.float32),
)


def kernel(Gu, Gi, edge_index, users, items,
           W1_0, W2_0, W1_1, W2_1, W1_2, W2_2):
    f32 = jnp.float32
    e0 = edge_index[0].astype(jnp.int32)
    e1 = edge_index[1].astype(jnp.int32)
    src2 = jnp.concatenate([e1 + NPH, e0])
    dst2 = jnp.concatenate([e0, e1])
    zeros_pad = jnp.zeros((RPW, KH), f32)

    npp = NU * KH // 128
    zp = jnp.zeros(((NPH - NU) * KH // 128, 128), f32)
    xa = jnp.concatenate([Gu[:, :KH].reshape(npp, 128), zp,
                          Gi[:, :KH].reshape(npp, 128), zp], axis=0)
    xb = jnp.concatenate([Gu[:, KH:].reshape(npp, 128), zp,
                          Gi[:, KH:].reshape(npp, 128), zp], axis=0)

    eye4 = jnp.eye(4, dtype=f32)

    def bd(m):
        return jnp.kron(eye4, m)

    onesbd = bd(jnp.ones((KH, KH), f32))

    tabs = [xa, xb]
    for (W1, W2) in ((W1_0, W2_0), (W1_1, W2_1), (W1_2, W2_2)):
        sa, sb = _spmv(src2, dst2, xa.reshape(NP, KH), xb.reshape(NP, KH),
                       zeros_pad)
        xa, xb = _dense(xa, xb, sa.reshape(PR, 128), sb.reshape(PR, 128),
                        bd(W1[:KH, :KH]), bd(W1[KH:, :KH]),
                        bd(W1[:KH, KH:]), bd(W1[KH:, KH:]),
                        bd(W2[:KH, :KH]), bd(W2[KH:, :KH]),
                        bd(W2[:KH, KH:]), bd(W2[KH:, KH:]), onesbd)
        tabs += [xa, xb]

    uid = users.astype(jnp.int32)
    iid = items.astype(jnp.int32) + NPH
    gamma_u, gamma_i = _gather(*[tb.reshape(NP, KH) for tb in tabs],
                               uid, iid)
    xui = _xui(gamma_u, gamma_i)
    return (xui, gamma_u, gamma_i)
